# baseline (speedup 1.0000x reference)
import jax
import jax.numpy as jnp
from jax import lax
from jax.experimental import pallas as pl
from jax.experimental.pallas import tpu as pltpu
from jax.experimental.pallas import tpu_sc as plsc

N_ROWS = 262144
NW = 32
ROWS_PER_W = N_ROWS // NW
NBUF = 7
CHUNK = 1024
N_CHUNKS = ROWS_PER_W // CHUNK
L = 16
COL0 = 32
NCOL = 16


def _rules(mi_x, su_x, mi_y, su_y):
    dist_x = jnp.abs(su_x - mi_x)
    dist_y = jnp.abs(su_y - mi_y)
    sx_gt = su_x > mi_x
    sx_lt = su_x < mi_x
    i3 = jnp.full((L,), 3, jnp.int32)
    i4 = jnp.full((L,), 4, jnp.int32)
    i2 = jnp.full((L,), 2, jnp.int32)
    i5 = jnp.full((L,), 5, jnp.int32)
    action = jnp.full((L,), 1, jnp.int32)
    action = jnp.where(dist_x < 23, jnp.where(sx_gt, i4, i3), action)
    action = jnp.where(dist_x > 27, jnp.where(sx_gt, i3, i4), action)
    action = jnp.where(dist_y > 4, jnp.where(su_y > mi_y, i5, i2), action)
    is_pl = (mi_x < 35) & sx_gt
    is_pr = (mi_x > 115) & sx_lt
    cond_pinned = (is_pl | is_pr) & (dist_x < 35)
    act_p_yc = jnp.where(mi_y < 50, i5, i2)
    act_p = jnp.where(dist_y <= 15, act_p_yc, jnp.where(is_pl, i3, i4))
    action = jnp.where(cond_pinned, act_p, action)
    return action


def _sc_body(ram_hbm, out_hbm, buf0, buf1, buf2, buf3, buf4, buf5, buf6,
             pres_v, sem0, sem1, sem2, sem3, sem4, sem5, sem6):
    wid = lax.axis_index("s") * 2 + lax.axis_index("c")
    base = wid * ROWS_PER_W
    bufs = (buf0, buf1, buf2, buf3, buf4, buf5, buf6)
    sems = (sem0, sem1, sem2, sem3, sem4, sem5, sem6)

    def start(ci):
        return pltpu.async_copy(
            ram_hbm.at[pl.ds(base + ci * CHUNK, CHUNK), pl.ds(COL0, NCOL)],
            bufs[ci % NBUF],
            sems[ci % NBUF],
        )

    lane = lax.iota(jnp.int32, L)

    acc = (jnp.zeros((L,), jnp.int32), jnp.zeros((L,), jnp.int32))
    copies = [None] * NBUF
    for ci in range(min(NBUF, N_CHUNKS)):
        copies[ci] = start(ci)
    for ci in range(N_CHUNKS):
        copies[ci % NBUF].wait()
        pend = ci - 1 + NBUF
        if ci >= 1 and pend < N_CHUNKS:
            copies[pend % NBUF] = start(pend)
        buf = bufs[ci % NBUF]

        def one(buf, ri):
            c0 = jnp.zeros((L,), jnp.int32)
            mi_x = plsc.load_gather(buf, [ri, c0])
            su_x = plsc.load_gather(buf, [ri, c0 + 1])
            mi_y = plsc.load_gather(buf, [ri, c0 + 2])
            su_y = plsc.load_gather(buf, [ri, c0 + 3])
            action = _rules(mi_x, su_x, mi_y, su_y)
            return jnp.int32(1) << (5 * action)

        def body(g, acc):
            ri = g * (2 * L) + lane
            return (acc[0] | one(buf, ri), acc[1] | one(buf, ri + L))

        acc = lax.fori_loop(0, CHUNK // (2 * L), body, acc)

    total = jnp.sum(acc[0] | acc[1])
    shift = jnp.minimum(5 * lane, 31)
    field = (jnp.full((L,), total) >> shift) & 31
    pres = jnp.where((lane < 6) & (field > 0), 1.0, 0.0).astype(jnp.float32)
    pres_v[...] = pres
    pltpu.sync_copy(pres_v, out_hbm.at[wid, pl.ds(0, L)])


@jax.jit
def kernel(ram):
    mesh = plsc.VectorSubcoreMesh(core_axis_name="c", subcore_axis_name="s")
    table = pl.kernel(
        _sc_body,
        out_type=jax.ShapeDtypeStruct((NW, 128), jnp.float32),
        mesh=mesh,
        scratch_types=(
            [pltpu.VMEM((CHUNK, NCOL), jnp.float32)] * NBUF
            + [pltpu.VMEM((L,), jnp.float32)]
            + [pltpu.SemaphoreType.DMA] * NBUF
        ),
        compiler_params=pltpu.CompilerParams(
            use_tc_tiling_on_sc=False, needs_layout_passes=False
        ),
    )(ram)

    def merge_body(t_ref, o_ref):
        t = lax.slice(t_ref[...], (0, 0), (NW, L))
        m = jnp.max(t, axis=0, keepdims=True)
        o_ref[...] = lax.slice(m, (0, 0), (1, 6))

    return pl.pallas_call(
        merge_body,
        out_shape=jax.ShapeDtypeStruct((1, 6), jnp.float32),
    )(table)


# speedup vs baseline: 1.0144x; 1.0144x over previous
import jax
import jax.numpy as jnp
from jax import lax
from jax.experimental import pallas as pl
from jax.experimental.pallas import tpu as pltpu
from jax.experimental.pallas import tpu_sc as plsc

N_ROWS = 262144
NW = 32
ROWS_PER_W = N_ROWS // NW
NBUF = 4
CHUNK = 1024
N_CHUNKS = ROWS_PER_W // CHUNK
L = 16
COL0 = 32
NCOL = 16


def _rules(mi_x, su_x, mi_y, su_y):
    dist_x = jnp.abs(su_x - mi_x)
    dist_y = jnp.abs(su_y - mi_y)
    sx_gt = su_x > mi_x
    sx_lt = su_x < mi_x
    i3 = jnp.full((L,), 3, jnp.int32)
    i4 = jnp.full((L,), 4, jnp.int32)
    i2 = jnp.full((L,), 2, jnp.int32)
    i5 = jnp.full((L,), 5, jnp.int32)
    action = jnp.full((L,), 1, jnp.int32)
    action = jnp.where(dist_x < 23, jnp.where(sx_gt, i4, i3), action)
    action = jnp.where(dist_x > 27, jnp.where(sx_gt, i3, i4), action)
    action = jnp.where(dist_y > 4, jnp.where(su_y > mi_y, i5, i2), action)
    is_pl = (mi_x < 35) & sx_gt
    is_pr = (mi_x > 115) & sx_lt
    cond_pinned = (is_pl | is_pr) & (dist_x < 35)
    act_p_yc = jnp.where(mi_y < 50, i5, i2)
    act_p = jnp.where(dist_y <= 15, act_p_yc, jnp.where(is_pl, i3, i4))
    action = jnp.where(cond_pinned, act_p, action)
    return action


def _sc_body(ram_hbm, out_hbm, *scratch):
    wid = lax.axis_index("s") * 2 + lax.axis_index("c")
    base = wid * ROWS_PER_W
    bufs = scratch[:NBUF]
    pres_v = scratch[NBUF]
    sems = scratch[NBUF + 1:]

    def start(ci):
        return pltpu.async_copy(
            ram_hbm.at[pl.ds(base + ci * CHUNK, CHUNK), pl.ds(COL0, NCOL)],
            bufs[ci % NBUF],
            sems[ci % NBUF],
        )

    lane = lax.iota(jnp.int32, L)

    acc = (jnp.zeros((L,), jnp.int32), jnp.zeros((L,), jnp.int32))
    copies = [None] * NBUF
    for ci in range(min(NBUF, N_CHUNKS)):
        copies[ci] = start(ci)
    for ci in range(N_CHUNKS):
        copies[ci % NBUF].wait()
        pend = ci - 1 + NBUF
        if ci >= 1 and pend < N_CHUNKS:
            copies[pend % NBUF] = start(pend)
        buf = bufs[ci % NBUF]

        def one(buf, ri):
            c0 = jnp.zeros((L,), jnp.int32)
            mi_x = plsc.load_gather(buf, [ri, c0])
            su_x = plsc.load_gather(buf, [ri, c0 + 1])
            mi_y = plsc.load_gather(buf, [ri, c0 + 2])
            su_y = plsc.load_gather(buf, [ri, c0 + 3])
            action = _rules(mi_x, su_x, mi_y, su_y)
            return jnp.int32(1) << (5 * action)

        def body(g, acc):
            ri = g * (2 * L) + lane
            return (acc[0] | one(buf, ri), acc[1] | one(buf, ri + L))

        acc = lax.fori_loop(0, CHUNK // (2 * L), body, acc)

    total = jnp.sum(acc[0] | acc[1])
    shift = jnp.minimum(5 * lane, 31)
    field = (jnp.full((L,), total) >> shift) & 31
    pres = jnp.where((lane < 6) & (field > 0), 1.0, 0.0).astype(jnp.float32)
    pres_v[...] = pres
    pltpu.sync_copy(pres_v, out_hbm.at[wid, pl.ds(0, L)])


@jax.jit
def kernel(ram):
    mesh = plsc.VectorSubcoreMesh(core_axis_name="c", subcore_axis_name="s")
    table = pl.kernel(
        _sc_body,
        out_type=jax.ShapeDtypeStruct((NW, 128), jnp.float32),
        mesh=mesh,
        scratch_types=(
            [pltpu.VMEM((CHUNK, NCOL), jnp.float32)] * NBUF
            + [pltpu.VMEM((L,), jnp.float32)]
            + [pltpu.SemaphoreType.DMA] * NBUF
        ),
        compiler_params=pltpu.CompilerParams(
            use_tc_tiling_on_sc=False, needs_layout_passes=False
        ),
    )(ram)

    def merge_body(t_ref, o_ref):
        t = lax.slice(t_ref[...], (0, 0), (NW, L))
        m = jnp.max(t, axis=0, keepdims=True)
        o_ref[...] = lax.slice(m, (0, 0), (1, 6))

    return pl.pallas_call(
        merge_body,
        out_shape=jax.ShapeDtypeStruct((1, 6), jnp.float32),
    )(table)


# speedup vs baseline: 1.0584x; 1.0434x over previous
import jax
import jax.numpy as jnp
from jax import lax
from jax.experimental import pallas as pl
from jax.experimental.pallas import tpu as pltpu
from jax.experimental.pallas import tpu_sc as plsc

N_ROWS = 262144
NW = 32
ROWS_PER_W = N_ROWS // NW
NBUF = 4
CHUNK = 512
N_CHUNKS = ROWS_PER_W // CHUNK
L = 16
COL0 = 32
NCOL = 16


def _rules(mi_x, su_x, mi_y, su_y):
    dist_x = jnp.abs(su_x - mi_x)
    dist_y = jnp.abs(su_y - mi_y)
    sx_gt = su_x > mi_x
    sx_lt = su_x < mi_x
    i3 = jnp.full((L,), 3, jnp.int32)
    i4 = jnp.full((L,), 4, jnp.int32)
    i2 = jnp.full((L,), 2, jnp.int32)
    i5 = jnp.full((L,), 5, jnp.int32)
    action = jnp.full((L,), 1, jnp.int32)
    action = jnp.where(dist_x < 23, jnp.where(sx_gt, i4, i3), action)
    action = jnp.where(dist_x > 27, jnp.where(sx_gt, i3, i4), action)
    action = jnp.where(dist_y > 4, jnp.where(su_y > mi_y, i5, i2), action)
    is_pl = (mi_x < 35) & sx_gt
    is_pr = (mi_x > 115) & sx_lt
    cond_pinned = (is_pl | is_pr) & (dist_x < 35)
    act_p_yc = jnp.where(mi_y < 50, i5, i2)
    act_p = jnp.where(dist_y <= 15, act_p_yc, jnp.where(is_pl, i3, i4))
    action = jnp.where(cond_pinned, act_p, action)
    return action


def _sc_body(ram_hbm, out_hbm, buf0, buf1, buf2, buf3, pres_v,
             sem0, sem1, sem2, sem3):
    wid = lax.axis_index("s") * 2 + lax.axis_index("c")
    base = wid * ROWS_PER_W
    bufs = (buf0, buf1, buf2, buf3)
    sems = (sem0, sem1, sem2, sem3)

    def start(ci):
        return pltpu.async_copy(
            ram_hbm.at[pl.ds(base + ci * CHUNK, CHUNK), pl.ds(COL0, NCOL)],
            bufs[ci % NBUF],
            sems[ci % NBUF],
        )

    lane = lax.iota(jnp.int32, L)

    acc = jnp.zeros((L,), jnp.int32)
    copies = [None] * NBUF
    for ci in range(NBUF):
        copies[ci] = start(ci)
    for ci in range(N_CHUNKS):
        copies[ci % NBUF].wait()
        buf = bufs[ci % NBUF]

        def body(g, acc):
            ri = g * L + lane
            c0 = jnp.zeros((L,), jnp.int32)
            mi_x = plsc.load_gather(buf, [ri, c0])
            su_x = plsc.load_gather(buf, [ri, c0 + 1])
            mi_y = plsc.load_gather(buf, [ri, c0 + 2])
            su_y = plsc.load_gather(buf, [ri, c0 + 3])
            action = _rules(mi_x, su_x, mi_y, su_y)
            return acc | (jnp.int32(1) << (5 * action))

        acc = lax.fori_loop(0, CHUNK // L, body, acc)
        if ci + NBUF < N_CHUNKS:
            copies[ci % NBUF] = start(ci + NBUF)

    total = jnp.sum(acc)
    shift = jnp.minimum(5 * lane, 31)
    field = (jnp.full((L,), total) >> shift) & 31
    pres = jnp.where((lane < 6) & (field > 0), 1.0, 0.0).astype(jnp.float32)
    pres_v[...] = pres
    pltpu.sync_copy(pres_v, out_hbm.at[wid, pl.ds(0, L)])


@jax.jit
def kernel(ram):
    mesh = plsc.VectorSubcoreMesh(core_axis_name="c", subcore_axis_name="s")
    table = pl.kernel(
        _sc_body,
        out_type=jax.ShapeDtypeStruct((NW, 128), jnp.float32),
        mesh=mesh,
        scratch_types=[
            pltpu.VMEM((CHUNK, NCOL), jnp.float32),
            pltpu.VMEM((CHUNK, NCOL), jnp.float32),
            pltpu.VMEM((CHUNK, NCOL), jnp.float32),
            pltpu.VMEM((CHUNK, NCOL), jnp.float32),
            pltpu.VMEM((L,), jnp.float32),
            pltpu.SemaphoreType.DMA,
            pltpu.SemaphoreType.DMA,
            pltpu.SemaphoreType.DMA,
            pltpu.SemaphoreType.DMA,
        ],
        compiler_params=pltpu.CompilerParams(
            use_tc_tiling_on_sc=False, needs_layout_passes=False
        ),
    )(ram)

    def merge_body(t_ref, o_ref):
        t = lax.slice(t_ref[...], (0, 0), (NW, L))
        m = jnp.max(t, axis=0, keepdims=True)
        o_ref[...] = lax.slice(m, (0, 0), (1, 6))

    return pl.pallas_call(
        merge_body,
        out_shape=jax.ShapeDtypeStruct((1, 6), jnp.float32),
    )(table)
